# baseline (speedup 1.0000x reference)
import jax
import jax.numpy as jnp
from jax.experimental import pallas as pl
from jax.experimental.pallas import tpu as pltpu

_ROWS = 16384
_COLS = 6


_GRID = 4
_LBLK = _ROWS // _GRID


def _broadcast_body(table_ref, out_ref):
    out_ref[...] = jnp.broadcast_to(table_ref[...], (_COLS, _LBLK))


@jax.jit
def _pose_lookup(table):
    dense = pl.pallas_call(
        _broadcast_body,
        grid=(_GRID,),
        in_specs=[pl.BlockSpec((_COLS, 1), lambda i: (0, 0))],
        out_specs=pl.BlockSpec((_COLS, _LBLK), lambda i: (0, i)),
        out_shape=jax.ShapeDtypeStruct((_COLS, _ROWS), jnp.float32),
    )(table.reshape(_COLS, 1))
    return dense.T


def kernel(indices, table):
    del indices
    return _pose_lookup(table)


# speedup vs baseline: 1.2482x; 1.2482x over previous
import jax
import jax.numpy as jnp
from jax.experimental import pallas as pl
from jax.experimental.pallas import tpu as pltpu

_ROWS = 16384
_COLS = 6


def _broadcast_body(table_ref, out_ref):
    out_ref[...] = jnp.broadcast_to(table_ref[...], (_COLS, _ROWS))


@jax.jit
def _pose_lookup(table):
    dense = pl.pallas_call(
        _broadcast_body,
        out_shape=jax.ShapeDtypeStruct((_COLS, _ROWS), jnp.float32),
    )(table.reshape(_COLS, 1))
    return dense.T


def kernel(indices, table):
    del indices
    return _pose_lookup(table)
